# baseline (speedup 1.0000x reference)
import functools

import jax
import jax.numpy as jnp
from jax import lax
from jax.experimental import pallas as pl
from jax.experimental.pallas import tpu as pltpu
from jax.experimental.pallas import tpu_sc as plsc

NUM_CORES = 2
NUM_SUBCORES = 16
NUM_WORKERS = NUM_CORES * NUM_SUBCORES
BATCH = 16384
DIM = 64
ROWS_PER_WORKER = BATCH // NUM_WORKERS
CHUNK = 128
CHUNKS_PER_WORKER = ROWS_PER_WORKER // CHUNK
NROWS = 1000000
QUAD_STRIDE = 1 << 18
QUAD_ROWS = QUAD_STRIDE

PACK_BLOCK = 8192


def _bf16_bits(x):
    xi = jax.lax.bitcast_convert_type(x, jnp.int32)
    rounded = xi + 0x7FFF + (jax.lax.shift_right_logical(xi, 16) & 1)
    return jax.lax.shift_right_logical(rounded, 16)


def _tc_pack_body(s0_ref, s1_ref, s2_ref, s3_ref, out_ref):
    groups = []
    for ref in (s0_ref, s1_ref, s2_ref, s3_ref):
        x = ref[...]
        lo = _bf16_bits(x[:32, :])
        hi = _bf16_bits(x[32:, :])
        w = lo | jax.lax.shift_left(hi, 16)
        groups.append(jnp.transpose(w, (1, 0)))
    out_ref[...] = jnp.concatenate(groups, axis=1)


def _tc_pack(tabT):
    grid = QUAD_ROWS // PACK_BLOCK
    nblk = QUAD_ROWS // PACK_BLOCK
    last_blk = (NROWS - 1) // PACK_BLOCK

    def slab(s):
        return pl.BlockSpec(
            (DIM, PACK_BLOCK),
            lambda i, s=s: (0, jnp.minimum(i + s * nblk, last_blk)))

    return pl.pallas_call(
        _tc_pack_body,
        grid=(grid,),
        in_specs=[slab(0), slab(1), slab(2), slab(3)],
        out_specs=pl.BlockSpec((PACK_BLOCK, 128), lambda i: (i, 0)),
        out_shape=jax.ShapeDtypeStruct((QUAD_ROWS, 128), jnp.int32),
    )(tabT, tabT, tabT, tabT)


def _sc_gather_body(uidx_hbm, midx_hbm, umf_hbm, mmf_hbm, umlp_hbm, mmlp_hbm,
                    umf_out, mmf_out, umlp_out, mmlp_out,
                    idx_u, idx_m, idx_u4, idx_m4,
                    buf_a, buf_b, buf_c, buf_d, sem):
    wid = lax.axis_index("s") * NUM_CORES + lax.axis_index("c")
    base = wid * ROWS_PER_WORKER

    pltpu.sync_copy(uidx_hbm.at[wid], idx_u)
    pltpu.sync_copy(midx_hbm.at[wid], idx_m)

    qmask = jnp.int32(QUAD_STRIDE - 1)
    for r in range(CHUNKS_PER_WORKER):
        for c in range(CHUNK // 16):
            sl = pl.ds(c * 16, 16)
            idx_u4[r, sl] = idx_u[r, sl] & qmask
            idx_m4[r, sl] = idx_m[r, sl] & qmask

    for k in range(CHUNKS_PER_WORKER):
        cps = [
            pltpu.async_copy(umf_hbm.at[idx_u4.at[k]], buf_a, sem),
            pltpu.async_copy(mmf_hbm.at[idx_m4.at[k]], buf_b, sem),
            pltpu.async_copy(umlp_hbm.at[idx_u4.at[k]], buf_c, sem),
            pltpu.async_copy(mmlp_hbm.at[idx_m4.at[k]], buf_d, sem),
        ]
        for cp in cps:
            cp.wait()
        orows = pl.ds(base + k * CHUNK, CHUNK)
        pltpu.sync_copy(buf_a, umf_out.at[orows])
        pltpu.sync_copy(buf_b, mmf_out.at[orows])
        pltpu.sync_copy(buf_c, umlp_out.at[orows])
        pltpu.sync_copy(buf_d, mmlp_out.at[orows])


_sc_gather = functools.partial(
    pl.kernel,
    mesh=plsc.VectorSubcoreMesh(core_axis_name="c", subcore_axis_name="s"),
    out_type=[jax.ShapeDtypeStruct((BATCH, 128), jnp.int32)] * 4,
    scratch_types=[
        pltpu.VMEM((CHUNKS_PER_WORKER, CHUNK), jnp.int32),
        pltpu.VMEM((CHUNKS_PER_WORKER, CHUNK), jnp.int32),
        pltpu.VMEM((CHUNKS_PER_WORKER, CHUNK), jnp.int32),
        pltpu.VMEM((CHUNKS_PER_WORKER, CHUNK), jnp.int32),
        pltpu.VMEM((CHUNK, 128), jnp.int32),
        pltpu.VMEM((CHUNK, 128), jnp.int32),
        pltpu.VMEM((CHUNK, 128), jnp.int32),
        pltpu.VMEM((CHUNK, 128), jnp.int32),
        pltpu.SemaphoreType.DMA,
    ],
    compiler_params=pltpu.CompilerParams(use_tc_tiling_on_sc=True),
)(_sc_gather_body)


TC_BLOCK = 2048


def _quad_unpack(quads, sel):
    a = jnp.where(sel < 2, quads[:, 0:32], quads[:, 64:96])
    b = jnp.where(sel < 2, quads[:, 32:64], quads[:, 96:128])
    g32 = jnp.where((sel & 1) == 0, a, b)
    lo_f = jax.lax.bitcast_convert_type(
        jax.lax.shift_left(g32, 16), jnp.float32)
    hi_f = jax.lax.bitcast_convert_type(
        g32 & jnp.int32(-65536), jnp.float32)
    return jnp.concatenate([lo_f, hi_f], axis=1)


def _tc_dense_body(umf_ref, mmf_ref, umlp_ref, mmlp_ref, usel_ref, msel_ref,
                   w1a_ref, w1b_ref, b1_ref, wf0_ref, wf1_ref, bf_ref, out_ref):
    usel = jax.lax.shift_right_logical(usel_ref[...], 18)
    msel = jax.lax.shift_right_logical(msel_ref[...], 18)
    u_mf = _quad_unpack(umf_ref[...], usel)
    m_mf = _quad_unpack(mmf_ref[...], msel)
    u_mlp = _quad_unpack(umlp_ref[...], usel)
    m_mlp = _quad_unpack(mmlp_ref[...], msel)
    h = jnp.dot(u_mlp, w1a_ref[...], preferred_element_type=jnp.float32)
    h = h + jnp.dot(m_mlp, w1b_ref[...], preferred_element_type=jnp.float32)
    h = jnp.maximum(h + b1_ref[...], 0.0)
    gmf = u_mf * m_mf
    logit = jnp.sum(gmf * wf0_ref[...], axis=1, keepdims=True)
    logit = logit + jnp.sum(h * wf1_ref[...], axis=1, keepdims=True)
    logit = logit + bf_ref[0, 0]
    out_ref[...] = jax.nn.sigmoid(logit)


def _tc_dense(umf, mmf, umlp, mmlp, usel, msel, w1a, w1b, b1, wf0, wf1, bf):
    grid = BATCH // TC_BLOCK
    row_spec = pl.BlockSpec((TC_BLOCK, 128), lambda i: (i, 0))
    sel_spec = pl.BlockSpec((TC_BLOCK, 1), lambda i: (i, 0))
    return pl.pallas_call(
        _tc_dense_body,
        grid=(grid,),
        in_specs=[row_spec, row_spec, row_spec, row_spec, sel_spec, sel_spec,
                  pl.BlockSpec((DIM, DIM), lambda i: (0, 0)),
                  pl.BlockSpec((DIM, DIM), lambda i: (0, 0)),
                  pl.BlockSpec((1, DIM), lambda i: (0, 0)),
                  pl.BlockSpec((1, DIM), lambda i: (0, 0)),
                  pl.BlockSpec((1, DIM), lambda i: (0, 0)),
                  pl.BlockSpec((1, 1), lambda i: (0, 0))],
        out_specs=pl.BlockSpec((TC_BLOCK, 1), lambda i: (i, 0)),
        out_shape=jax.ShapeDtypeStruct((BATCH, 1), jnp.float32),
    )(umf, mmf, umlp, mmlp, usel, msel, w1a, w1b, b1, wf0, wf1, bf)


def kernel(x, user_mf, movie_mf, user_mlp, movie_mlp, W1, b1, Wf, bf):
    u_idx = x[:, 0]
    m_idx = x[:, 1]
    u_idx3 = u_idx.reshape(NUM_WORKERS, CHUNKS_PER_WORKER, CHUNK)
    m_idx3 = m_idx.reshape(NUM_WORKERS, CHUNKS_PER_WORKER, CHUNK)
    umf_packed = _tc_pack(user_mf.T)
    mmf_packed = _tc_pack(movie_mf.T)
    umlp_packed = _tc_pack(user_mlp.T)
    mmlp_packed = _tc_pack(movie_mlp.T)
    umf_pairs, mmf_pairs, umlp_quads, mmlp_quads = _sc_gather(
        u_idx3, m_idx3, umf_packed, mmf_packed, umlp_packed, mmlp_packed)
    usel = u_idx.reshape(BATCH, 1)
    msel = m_idx.reshape(BATCH, 1)
    return _tc_dense(umf_pairs, mmf_pairs, umlp_quads, mmlp_quads, usel, msel,
                     W1[:DIM], W1[DIM:], b1.reshape(1, DIM),
                     Wf[:DIM].reshape(1, DIM), Wf[DIM:].reshape(1, DIM),
                     bf.reshape(1, 1))


# speedup vs baseline: 1.0115x; 1.0115x over previous
import functools

import jax
import jax.numpy as jnp
from jax import lax
from jax.experimental import pallas as pl
from jax.experimental.pallas import tpu as pltpu
from jax.experimental.pallas import tpu_sc as plsc

NUM_CORES = 2
NUM_SUBCORES = 16
NUM_WORKERS = NUM_CORES * NUM_SUBCORES
BATCH = 16384
DIM = 64
ROWS_PER_WORKER = BATCH // NUM_WORKERS
CHUNK = 128
CHUNKS_PER_WORKER = ROWS_PER_WORKER // CHUNK
NROWS = 1000000
QUAD_STRIDE = 1 << 18
QUAD_ROWS = QUAD_STRIDE

PACK_BLOCK = 8192


def _bf16_bits(x):
    xi = jax.lax.bitcast_convert_type(x, jnp.int32)
    rounded = xi + 0x7FFF + (jax.lax.shift_right_logical(xi, 16) & 1)
    return jax.lax.shift_right_logical(rounded, 16)


def _tc_pack_body(s0_ref, s1_ref, s2_ref, s3_ref, out_ref):
    groups = []
    for ref in (s0_ref, s1_ref, s2_ref, s3_ref):
        x = ref[...]
        lo = _bf16_bits(x[:32, :])
        hi = _bf16_bits(x[32:, :])
        w = lo | jax.lax.shift_left(hi, 16)
        groups.append(jnp.transpose(w, (1, 0)))
    out_ref[...] = jnp.concatenate(groups, axis=1)


def _tc_pack(tabT):
    grid = QUAD_ROWS // PACK_BLOCK
    nblk = QUAD_ROWS // PACK_BLOCK
    last_blk = (NROWS - 1) // PACK_BLOCK

    def slab(s):
        return pl.BlockSpec(
            (DIM, PACK_BLOCK),
            lambda i, s=s: (0, jnp.minimum(i + s * nblk, last_blk)))

    return pl.pallas_call(
        _tc_pack_body,
        grid=(grid,),
        in_specs=[slab(0), slab(1), slab(2), slab(3)],
        out_specs=pl.BlockSpec((PACK_BLOCK, 128), lambda i: (i, 0)),
        out_shape=jax.ShapeDtypeStruct((QUAD_ROWS, 128), jnp.int32),
    )(tabT, tabT, tabT, tabT)


def _sc_gather_body(idx_hbm, mf_hbm, mlp_hbm, mf_out, mlp_out,
                    idx_v, idx_q, buf_a, buf_b, sem):
    wid = lax.axis_index("s") * NUM_CORES + lax.axis_index("c")
    base = wid * ROWS_PER_WORKER

    pltpu.sync_copy(idx_hbm.at[wid], idx_v)

    qmask = jnp.int32(QUAD_STRIDE - 1)
    for r in range(CHUNKS_PER_WORKER):
        for c in range(CHUNK // 16):
            sl = pl.ds(c * 16, 16)
            idx_q[r, sl] = idx_v[r, sl] & qmask

    for k in range(CHUNKS_PER_WORKER):
        cps = [
            pltpu.async_copy(mf_hbm.at[idx_q.at[k]], buf_a, sem),
            pltpu.async_copy(mlp_hbm.at[idx_q.at[k]], buf_b, sem),
        ]
        for cp in cps:
            cp.wait()
        orows = pl.ds(base + k * CHUNK, CHUNK)
        pltpu.sync_copy(buf_a, mf_out.at[orows])
        pltpu.sync_copy(buf_b, mlp_out.at[orows])


_sc_gather = functools.partial(
    pl.kernel,
    mesh=plsc.VectorSubcoreMesh(core_axis_name="c", subcore_axis_name="s"),
    out_type=[jax.ShapeDtypeStruct((BATCH, 128), jnp.int32)] * 2,
    scratch_types=[
        pltpu.VMEM((CHUNKS_PER_WORKER, CHUNK), jnp.int32),
        pltpu.VMEM((CHUNKS_PER_WORKER, CHUNK), jnp.int32),
        pltpu.VMEM((CHUNK, 128), jnp.int32),
        pltpu.VMEM((CHUNK, 128), jnp.int32),
        pltpu.SemaphoreType.DMA,
    ],
    compiler_params=pltpu.CompilerParams(use_tc_tiling_on_sc=True),
)(_sc_gather_body)


TC_BLOCK = 2048


def _quad_unpack(quads, sel):
    a = jnp.where(sel < 2, quads[:, 0:32], quads[:, 64:96])
    b = jnp.where(sel < 2, quads[:, 32:64], quads[:, 96:128])
    g32 = jnp.where((sel & 1) == 0, a, b)
    lo_f = jax.lax.bitcast_convert_type(
        jax.lax.shift_left(g32, 16), jnp.float32)
    hi_f = jax.lax.bitcast_convert_type(
        g32 & jnp.int32(-65536), jnp.float32)
    return jnp.concatenate([lo_f, hi_f], axis=1)


def _tc_dense_body(umf_ref, mmf_ref, umlp_ref, mmlp_ref, usel_ref, msel_ref,
                   w1a_ref, w1b_ref, b1_ref, wf0_ref, wf1_ref, bf_ref, out_ref):
    usel = jax.lax.shift_right_logical(usel_ref[...], 18)
    msel = jax.lax.shift_right_logical(msel_ref[...], 18)
    u_mf = _quad_unpack(umf_ref[...], usel)
    m_mf = _quad_unpack(mmf_ref[...], msel)
    u_mlp = _quad_unpack(umlp_ref[...], usel)
    m_mlp = _quad_unpack(mmlp_ref[...], msel)
    h = jnp.dot(u_mlp, w1a_ref[...], preferred_element_type=jnp.float32)
    h = h + jnp.dot(m_mlp, w1b_ref[...], preferred_element_type=jnp.float32)
    h = jnp.maximum(h + b1_ref[...], 0.0)
    gmf = u_mf * m_mf
    logit = jnp.sum(gmf * wf0_ref[...], axis=1, keepdims=True)
    logit = logit + jnp.sum(h * wf1_ref[...], axis=1, keepdims=True)
    logit = logit + bf_ref[0, 0]
    out_ref[...] = jax.nn.sigmoid(logit)


def _tc_dense(umf, mmf, umlp, mmlp, usel, msel, w1a, w1b, b1, wf0, wf1, bf):
    grid = BATCH // TC_BLOCK
    row_spec = pl.BlockSpec((TC_BLOCK, 128), lambda i: (i, 0))
    sel_spec = pl.BlockSpec((TC_BLOCK, 1), lambda i: (i, 0))
    return pl.pallas_call(
        _tc_dense_body,
        grid=(grid,),
        in_specs=[row_spec, row_spec, row_spec, row_spec, sel_spec, sel_spec,
                  pl.BlockSpec((DIM, DIM), lambda i: (0, 0)),
                  pl.BlockSpec((DIM, DIM), lambda i: (0, 0)),
                  pl.BlockSpec((1, DIM), lambda i: (0, 0)),
                  pl.BlockSpec((1, DIM), lambda i: (0, 0)),
                  pl.BlockSpec((1, DIM), lambda i: (0, 0)),
                  pl.BlockSpec((1, 1), lambda i: (0, 0))],
        out_specs=pl.BlockSpec((TC_BLOCK, 1), lambda i: (i, 0)),
        out_shape=jax.ShapeDtypeStruct((BATCH, 1), jnp.float32),
    )(umf, mmf, umlp, mmlp, usel, msel, w1a, w1b, b1, wf0, wf1, bf)


def kernel(x, user_mf, movie_mf, user_mlp, movie_mlp, W1, b1, Wf, bf):
    u_idx = x[:, 0]
    m_idx = x[:, 1]
    u_idx3 = u_idx.reshape(NUM_WORKERS, CHUNKS_PER_WORKER, CHUNK)
    m_idx3 = m_idx.reshape(NUM_WORKERS, CHUNKS_PER_WORKER, CHUNK)
    umf_packed = _tc_pack(user_mf.T)
    umlp_packed = _tc_pack(user_mlp.T)
    umf_pairs, umlp_quads = _sc_gather(u_idx3, umf_packed, umlp_packed)
    mmf_packed = _tc_pack(movie_mf.T)
    mmlp_packed = _tc_pack(movie_mlp.T)
    mmf_pairs, mmlp_quads = _sc_gather(m_idx3, mmf_packed, mmlp_packed)
    usel = u_idx.reshape(BATCH, 1)
    msel = m_idx.reshape(BATCH, 1)
    return _tc_dense(umf_pairs, mmf_pairs, umlp_quads, mmlp_quads, usel, msel,
                     W1[:DIM], W1[DIM:], b1.reshape(1, DIM),
                     Wf[:DIM].reshape(1, DIM), Wf[DIM:].reshape(1, DIM),
                     bf.reshape(1, 1))
